# Initial kernel scaffold; baseline (speedup 1.0000x reference)
import functools

import jax
import jax.numpy as jnp
from jax import lax
from jax.experimental import pallas as pl
from jax.experimental.pallas import tpu as pltpu
from jax.experimental.pallas import tpu_sc as plsc

_N = 10000
_E = 320000
_ND = 128
_HD = 256
_ED = 16
_NRBF = 16
_CUTOFF = 10.0
_XP = 16

_CH = 128
_NCHUNK = _E // _CH
_NC = 2
_NS = 16
_NW = _NC * _NS
_NPS = _N // _NS

_BE = 1280
_BN = 1000

_mesh = plsc.VectorSubcoreMesh(
    core_axis_name="c", subcore_axis_name="s", num_cores=_NC, num_subcores=_NS
)


def _silu(v):
    return v * jax.nn.sigmoid(v)


def _sc_gather_body(h_hbm, xp_hbm, ii_hbm, jj_hbm,
                    hi_hbm, hj_hbm, xi_hbm, xj_hbm,
                    idx_i, idx_j, bhi, bhj, bxi, bxj, sem):
    c = lax.axis_index("c")
    s = lax.axis_index("s")
    wid = s * _NC + c
    trips = (_NCHUNK + _NW - 1) // _NW

    def body(t, carry):
        chunk = wid + t * _NW

        @pl.when(chunk < _NCHUNK)
        def _():
            base = chunk * _CH
            pltpu.sync_copy(ii_hbm.at[pl.ds(base, _CH)], idx_i)
            pltpu.sync_copy(jj_hbm.at[pl.ds(base, _CH)], idx_j)
            c1 = pltpu.async_copy(h_hbm.at[idx_i], bhi, sem)
            c2 = pltpu.async_copy(h_hbm.at[idx_j], bhj, sem)
            c3 = pltpu.async_copy(xp_hbm.at[idx_i], bxi, sem)
            c4 = pltpu.async_copy(xp_hbm.at[idx_j], bxj, sem)
            c1.wait()
            c2.wait()
            c3.wait()
            c4.wait()
            pltpu.sync_copy(bhi, hi_hbm.at[pl.ds(base, _CH)])
            pltpu.sync_copy(bhj, hj_hbm.at[pl.ds(base, _CH)])
            pltpu.sync_copy(bxi, xi_hbm.at[pl.ds(base, _CH)])
            pltpu.sync_copy(bxj, xj_hbm.at[pl.ds(base, _CH)])

        return carry

    lax.fori_loop(0, trips, body, 0)


_gather_call = pl.kernel(
    _sc_gather_body,
    out_type=(
        jax.ShapeDtypeStruct((_E, _ND), jnp.float32),
        jax.ShapeDtypeStruct((_E, _ND), jnp.float32),
        jax.ShapeDtypeStruct((_E, _XP), jnp.float32),
        jax.ShapeDtypeStruct((_E, _XP), jnp.float32),
    ),
    mesh=_mesh,
    scratch_types=[
        pltpu.VMEM((_CH,), jnp.int32),
        pltpu.VMEM((_CH,), jnp.int32),
        pltpu.VMEM((_CH, _ND), jnp.float32),
        pltpu.VMEM((_CH, _ND), jnp.float32),
        pltpu.VMEM((_CH, _XP), jnp.float32),
        pltpu.VMEM((_CH, _XP), jnp.float32),
        pltpu.SemaphoreType.DMA,
    ],
)


def _sc_scatter_body(mlo_hbm, mhi_hbm, wd_hbm, ii_hbm, z_hbm, zx_hbm,
                     silo_hbm, sihi_hbm, xacc_hbm,
                     idx_v, mbuf, wbuf, shm, shx, sem):
    c = lax.axis_index("c")
    s = lax.axis_index("s")
    rows = pl.ds(s * _NPS, _NPS)
    pltpu.sync_copy(z_hbm, shm.at[rows])
    pltpu.sync_copy(zx_hbm, shx.at[rows])
    plsc.subcore_barrier()

    trips = (_NCHUNK + _NS - 1) // _NS

    def body(t, carry):
        chunk = s + t * _NS

        @pl.when(chunk < _NCHUNK)
        def _():
            base = chunk * _CH
            pltpu.sync_copy(ii_hbm.at[pl.ds(base, _CH)], idx_v)

            @pl.when(c == 0)
            def _():
                pltpu.sync_copy(mlo_hbm.at[pl.ds(base, _CH)], mbuf)
                pltpu.sync_copy(wd_hbm.at[pl.ds(base, _CH)], wbuf)
                pltpu.sync_copy(mbuf, shm.at[idx_v], add=True)
                pltpu.sync_copy(wbuf, shx.at[idx_v], add=True)

            @pl.when(c == 1)
            def _():
                pltpu.sync_copy(mhi_hbm.at[pl.ds(base, _CH)], mbuf)
                pltpu.sync_copy(mbuf, shm.at[idx_v], add=True)

        return carry

    lax.fori_loop(0, trips, body, 0)
    plsc.subcore_barrier()

    @pl.when(c == 0)
    def _():
        pltpu.sync_copy(shm.at[rows], silo_hbm.at[rows])
        pltpu.sync_copy(shx.at[rows], xacc_hbm.at[rows])

    @pl.when(c == 1)
    def _():
        pltpu.sync_copy(shm.at[rows], sihi_hbm.at[rows])


_scatter_call = pl.kernel(
    _sc_scatter_body,
    out_type=(
        jax.ShapeDtypeStruct((_N, _ND), jnp.float32),
        jax.ShapeDtypeStruct((_N, _ND), jnp.float32),
        jax.ShapeDtypeStruct((_N, _XP), jnp.float32),
    ),
    mesh=_mesh,
    scratch_types=[
        pltpu.VMEM((_CH,), jnp.int32),
        pltpu.VMEM((_CH, _ND), jnp.float32),
        pltpu.VMEM((_CH, _XP), jnp.float32),
        pltpu.VMEM_SHARED((_N, _ND), jnp.float32),
        pltpu.VMEM_SHARED((_N, _XP), jnp.float32),
        pltpu.SemaphoreType.DMA,
    ],
)


def _tc_edge_body(hi, hj, xi, xj, ea,
                  w1hi, w1hj, w1r, w1e, b1, w2, b2, wx1, bx1, wx2,
                  mlo_o, mhi_o, wd_o):
    f32 = jnp.float32
    di = xi[...] - xj[...]
    d2 = jnp.sum(di * di, axis=1, keepdims=True) + 1e-8
    dist = jnp.sqrt(d2)
    centers = lax.broadcasted_iota(f32, (1, _NRBF), 1) * (_CUTOFF / (_NRBF - 1))
    zz = (dist - centers) * (_NRBF / _CUTOFF)
    rbf = jnp.exp(-0.5 * zz * zz)
    pre = (jnp.dot(hi[...], w1hi[...], preferred_element_type=f32)
           + jnp.dot(hj[...], w1hj[...], preferred_element_type=f32)
           + jnp.dot(rbf, w1r[...], preferred_element_type=f32)
           + jnp.dot(ea[...], w1e[...], preferred_element_type=f32)
           + b1[...])
    m = _silu(pre)
    m = _silu(jnp.dot(m, w2[...], preferred_element_type=f32) + b2[...])
    t = _silu(jnp.dot(m, wx1[...], preferred_element_type=f32) + bx1[...])
    cw = jnp.dot(t, wx2[...], preferred_element_type=f32)
    mlo_o[...] = m[:, :_ND]
    mhi_o[...] = m[:, _ND:]
    wd_o[...] = di * cw


def _edge_call(hi, hj, xi, xj, ea, w1hi, w1hj, w1r, w1e, b1, w2, b2, wx1, bx1, wx2):
    grid = (_E // _BE,)
    row = lambda i: (i, 0)
    full = lambda i: (0, 0)
    return pl.pallas_call(
        _tc_edge_body,
        grid=grid,
        in_specs=[
            pl.BlockSpec((_BE, _ND), row),
            pl.BlockSpec((_BE, _ND), row),
            pl.BlockSpec((_BE, _XP), row),
            pl.BlockSpec((_BE, _XP), row),
            pl.BlockSpec((_BE, _ED), row),
            pl.BlockSpec((_ND, _HD), full),
            pl.BlockSpec((_ND, _HD), full),
            pl.BlockSpec((_NRBF, _HD), full),
            pl.BlockSpec((_ED, _HD), full),
            pl.BlockSpec((1, _HD), full),
            pl.BlockSpec((_HD, _HD), full),
            pl.BlockSpec((1, _HD), full),
            pl.BlockSpec((_HD, _HD), full),
            pl.BlockSpec((1, _HD), full),
            pl.BlockSpec((_HD, 1), full),
        ],
        out_specs=[
            pl.BlockSpec((_BE, _ND), row),
            pl.BlockSpec((_BE, _ND), row),
            pl.BlockSpec((_BE, _XP), row),
        ],
        out_shape=[
            jax.ShapeDtypeStruct((_E, _ND), jnp.float32),
            jax.ShapeDtypeStruct((_E, _ND), jnp.float32),
            jax.ShapeDtypeStruct((_E, _XP), jnp.float32),
        ],
        compiler_params=pltpu.CompilerParams(
            dimension_semantics=("arbitrary",),
        ),
    )(hi, hj, xi, xj, ea, w1hi, w1hj, w1r, w1e, b1, w2, b2, wx1, bx1, wx2)


def _tc_node_body(h, mlo, mhi, xp, xacc,
                  w1h, w1lo, w1hi_, bh1, wh2, bh2, g, b,
                  hn_o, xp_o):
    f32 = jnp.float32
    hv = h[...]
    pre = (jnp.dot(hv, w1h[...], preferred_element_type=f32)
           + jnp.dot(mlo[...], w1lo[...], preferred_element_type=f32)
           + jnp.dot(mhi[...], w1hi_[...], preferred_element_type=f32)
           + bh1[...])
    u = jnp.dot(_silu(pre), wh2[...], preferred_element_type=f32) + bh2[...]
    hn = hv + u
    mu = jnp.mean(hn, axis=1, keepdims=True)
    var = jnp.mean((hn - mu) * (hn - mu), axis=1, keepdims=True)
    hn_o[...] = (hn - mu) * lax.rsqrt(var + 1e-5) * g[...] + b[...]
    xp_o[...] = xp[...] + xacc[...]


def _node_call(h, mlo, mhi, xp, xacc, w1h, w1lo, w1hi_, bh1, wh2, bh2, g, b):
    grid = (_N // _BN,)
    row = lambda i: (i, 0)
    full = lambda i: (0, 0)
    return pl.pallas_call(
        _tc_node_body,
        grid=grid,
        in_specs=[
            pl.BlockSpec((_BN, _ND), row),
            pl.BlockSpec((_BN, _ND), row),
            pl.BlockSpec((_BN, _ND), row),
            pl.BlockSpec((_BN, _XP), row),
            pl.BlockSpec((_BN, _XP), row),
            pl.BlockSpec((_ND, _HD), full),
            pl.BlockSpec((_ND, _HD), full),
            pl.BlockSpec((_ND, _HD), full),
            pl.BlockSpec((1, _HD), full),
            pl.BlockSpec((_HD, _ND), full),
            pl.BlockSpec((1, _ND), full),
            pl.BlockSpec((1, _ND), full),
            pl.BlockSpec((1, _ND), full),
        ],
        out_specs=[
            pl.BlockSpec((_BN, _ND), row),
            pl.BlockSpec((_BN, _XP), row),
        ],
        out_shape=[
            jax.ShapeDtypeStruct((_N, _ND), jnp.float32),
            jax.ShapeDtypeStruct((_N, _XP), jnp.float32),
        ],
        compiler_params=pltpu.CompilerParams(
            dimension_semantics=("arbitrary",),
        ),
    )(h, mlo, mhi, xp, xacc, w1h, w1lo, w1hi_, bh1, wh2, bh2, g, b)


def kernel(h, x, edge_index, edge_attr, params):
    ei = edge_index.astype(jnp.int32)
    ii = ei[1]
    jj = ei[0]
    xp = jnp.pad(x.astype(jnp.float32), ((0, 0), (0, _XP - 3)))
    z = jnp.zeros((_NPS, _ND), jnp.float32)
    zx = jnp.zeros((_NPS, _XP), jnp.float32)
    for p in params:
        hi, hj, xi, xj = _gather_call(h, xp, ii, jj)
        w1 = p["e1"]["w"]
        mlo, mhi, wd = _edge_call(
            hi, hj, xi, xj, edge_attr,
            w1[:_ND], w1[_ND:2 * _ND], w1[2 * _ND:2 * _ND + _NRBF],
            w1[2 * _ND + _NRBF:], p["e1"]["b"][None],
            p["e2"]["w"], p["e2"]["b"][None],
            p["x1"]["w"], p["x1"]["b"][None], p["x2"]["w"],
        )
        silo, sihi, xacc = _scatter_call(mlo, mhi, wd, ii, z, zx)
        wh1 = p["h1"]["w"]
        h, xp = _node_call(
            h, silo, sihi, xp, xacc,
            wh1[:_ND], wh1[_ND:2 * _ND], wh1[2 * _ND:], p["h1"]["b"][None],
            p["h2"]["w"], p["h2"]["b"][None], p["ln_g"][None], p["ln_b"][None],
        )
    return (h, xp[:, :3])


# speedup vs baseline: 2.1076x; 2.1076x over previous
import functools

import jax
import jax.numpy as jnp
from jax import lax
from jax.experimental import pallas as pl
from jax.experimental.pallas import tpu as pltpu
from jax.experimental.pallas import tpu_sc as plsc

_N = 10000
_E = 320000
_ND = 128
_HD = 256
_ED = 16
_NRBF = 16
_CUTOFF = 10.0
_XP = 16

_CH = 128
_NCHUNK = _E // _CH
_NC = 2
_NS = 16
_NW = _NC * _NS
_NPS = _N // _NS

_BE = 1280
_BN = 1000

@functools.lru_cache(maxsize=None)
def _sc_mesh():
    return plsc.VectorSubcoreMesh(
        core_axis_name="c", subcore_axis_name="s", num_cores=_NC, num_subcores=_NS
    )


def _silu(v):
    return v * jax.nn.sigmoid(v)


def _sc_gather_body(h_hbm, xp_hbm, ii_hbm, jj_hbm,
                    hi_hbm, hj_hbm, xi_hbm, xj_hbm,
                    idx_i, idx_j, bhi, bhj, bxi, bxj, sem):
    c = lax.axis_index("c")
    s = lax.axis_index("s")
    wid = s * _NC + c
    trips = (_NCHUNK + _NW - 1) // _NW

    def body(t, carry):
        chunk = wid + t * _NW

        @pl.when(chunk < _NCHUNK)
        def _():
            base = chunk * _CH
            pltpu.sync_copy(ii_hbm.at[pl.ds(base, _CH)], idx_i)
            pltpu.sync_copy(jj_hbm.at[pl.ds(base, _CH)], idx_j)
            c1 = pltpu.async_copy(h_hbm.at[idx_i], bhi, sem)
            c2 = pltpu.async_copy(h_hbm.at[idx_j], bhj, sem)
            c3 = pltpu.async_copy(xp_hbm.at[idx_i], bxi, sem)
            c4 = pltpu.async_copy(xp_hbm.at[idx_j], bxj, sem)
            c1.wait()
            c2.wait()
            c3.wait()
            c4.wait()
            pltpu.sync_copy(bhi, hi_hbm.at[pl.ds(base, _CH)])
            pltpu.sync_copy(bhj, hj_hbm.at[pl.ds(base, _CH)])
            pltpu.sync_copy(bxi, xi_hbm.at[pl.ds(base, _CH)])
            pltpu.sync_copy(bxj, xj_hbm.at[pl.ds(base, _CH)])

        return carry

    lax.fori_loop(0, trips, body, 0)


@functools.lru_cache(maxsize=None)
def _gather_kernel():
    return pl.kernel(
        _sc_gather_body,
        out_type=(
            jax.ShapeDtypeStruct((_E, _ND), jnp.float32),
            jax.ShapeDtypeStruct((_E, _ND), jnp.float32),
            jax.ShapeDtypeStruct((_E, _XP), jnp.float32),
            jax.ShapeDtypeStruct((_E, _XP), jnp.float32),
        ),
        mesh=_sc_mesh(),
        compiler_params=pltpu.CompilerParams(use_tc_tiling_on_sc=False),
        scratch_types=[
            pltpu.VMEM((_CH,), jnp.int32),
            pltpu.VMEM((_CH,), jnp.int32),
            pltpu.VMEM((_CH, _ND), jnp.float32),
            pltpu.VMEM((_CH, _ND), jnp.float32),
            pltpu.VMEM((_CH, _XP), jnp.float32),
            pltpu.VMEM((_CH, _XP), jnp.float32),
            pltpu.SemaphoreType.DMA,
        ],
    )


def _gather_call(h, xp, ii, jj):
    return _gather_kernel()(h, xp, ii, jj)


def _sc_scatter_body(mlo_hbm, mhi_hbm, wd_hbm, ii_hbm, z_hbm, zx_hbm,
                     silo_hbm, sihi_hbm, xacc_hbm,
                     idx_v, mbuf, wbuf, shm, shx, sem):
    c = lax.axis_index("c")
    s = lax.axis_index("s")
    rows = pl.ds(s * _NPS, _NPS)
    pltpu.sync_copy(z_hbm, shm.at[rows])
    pltpu.sync_copy(zx_hbm, shx.at[rows])
    plsc.subcore_barrier()

    trips = (_NCHUNK + _NS - 1) // _NS

    def body(t, carry):
        chunk = s + t * _NS

        @pl.when(chunk < _NCHUNK)
        def _():
            base = chunk * _CH
            pltpu.sync_copy(ii_hbm.at[pl.ds(base, _CH)], idx_v)

            @pl.when(c == 0)
            def _():
                pltpu.sync_copy(mlo_hbm.at[pl.ds(base, _CH)], mbuf)
                pltpu.sync_copy(wd_hbm.at[pl.ds(base, _CH)], wbuf)
                pltpu.sync_copy(mbuf, shm.at[idx_v], add=True)
                pltpu.sync_copy(wbuf, shx.at[idx_v], add=True)

            @pl.when(c == 1)
            def _():
                pltpu.sync_copy(mhi_hbm.at[pl.ds(base, _CH)], mbuf)
                pltpu.sync_copy(mbuf, shm.at[idx_v], add=True)

        return carry

    lax.fori_loop(0, trips, body, 0)
    plsc.subcore_barrier()

    @pl.when(c == 0)
    def _():
        pltpu.sync_copy(shm.at[rows], silo_hbm.at[rows])
        pltpu.sync_copy(shx.at[rows], xacc_hbm.at[rows])

    @pl.when(c == 1)
    def _():
        pltpu.sync_copy(shm.at[rows], sihi_hbm.at[rows])


@functools.lru_cache(maxsize=None)
def _scatter_kernel():
    return pl.kernel(
        _sc_scatter_body,
        out_type=(
            jax.ShapeDtypeStruct((_N, _ND), jnp.float32),
            jax.ShapeDtypeStruct((_N, _ND), jnp.float32),
            jax.ShapeDtypeStruct((_N, _XP), jnp.float32),
        ),
        mesh=_sc_mesh(),
        compiler_params=pltpu.CompilerParams(use_tc_tiling_on_sc=False),
        scratch_types=[
            pltpu.VMEM((_CH,), jnp.int32),
            pltpu.VMEM((_CH, _ND), jnp.float32),
            pltpu.VMEM((_CH, _XP), jnp.float32),
            pltpu.VMEM_SHARED((_N, _ND), jnp.float32),
            pltpu.VMEM_SHARED((_N, _XP), jnp.float32),
            pltpu.SemaphoreType.DMA,
        ],
    )


def _scatter_call(mlo, mhi, wd, ii, z, zx):
    return _scatter_kernel()(mlo, mhi, wd, ii, z, zx)


def _tc_edge_body(hi, hj, xi, xj, ea,
                  w1, b1, w2, b2, wx1, bx1, wx2,
                  mlo_o, mhi_o, wd_o):
    f32 = jnp.float32
    di = xi[...] - xj[...]
    d2 = jnp.sum(di * di, axis=1, keepdims=True) + 1e-8
    dist = jnp.sqrt(d2)
    centers = lax.broadcasted_iota(jnp.int32, (1, _NRBF), 1).astype(f32) * (
        _CUTOFF / (_NRBF - 1))
    zz = (dist - centers) * (_NRBF / _CUTOFF)
    rbf = jnp.exp(-0.5 * zz * zz)
    msg = jnp.concatenate([hi[...], hj[...], rbf, ea[...]], axis=1)
    pre = jnp.dot(msg, w1[...], preferred_element_type=f32) + b1[...]
    m = _silu(pre)
    m = _silu(jnp.dot(m, w2[...], preferred_element_type=f32) + b2[...])
    t = _silu(jnp.dot(m, wx1[...], preferred_element_type=f32) + bx1[...])
    cw = jnp.dot(t, wx2[...], preferred_element_type=f32)
    mlo_o[...] = m[:, :_ND]
    mhi_o[...] = m[:, _ND:]
    wd_o[...] = di * cw


def _edge_call(hi, hj, xi, xj, ea, w1, b1, w2, b2, wx1, bx1, wx2):
    grid = (_E // _BE,)
    row = lambda i: (i, 0)
    full = lambda i: (0, 0)
    return pl.pallas_call(
        _tc_edge_body,
        grid=grid,
        in_specs=[
            pl.BlockSpec((_BE, _ND), row),
            pl.BlockSpec((_BE, _ND), row),
            pl.BlockSpec((_BE, _XP), row),
            pl.BlockSpec((_BE, _XP), row),
            pl.BlockSpec((_BE, _ED), row),
            pl.BlockSpec((2 * _ND + _NRBF + _ED, _HD), full),
            pl.BlockSpec((1, _HD), full),
            pl.BlockSpec((_HD, _HD), full),
            pl.BlockSpec((1, _HD), full),
            pl.BlockSpec((_HD, _HD), full),
            pl.BlockSpec((1, _HD), full),
            pl.BlockSpec((_HD, 1), full),
        ],
        out_specs=[
            pl.BlockSpec((_BE, _ND), row),
            pl.BlockSpec((_BE, _ND), row),
            pl.BlockSpec((_BE, _XP), row),
        ],
        out_shape=[
            jax.ShapeDtypeStruct((_E, _ND), jnp.float32),
            jax.ShapeDtypeStruct((_E, _ND), jnp.float32),
            jax.ShapeDtypeStruct((_E, _XP), jnp.float32),
        ],
        compiler_params=pltpu.CompilerParams(
            dimension_semantics=("arbitrary",),
        ),
    )(hi, hj, xi, xj, ea, w1, b1, w2, b2, wx1, bx1, wx2)


def _tc_node_body(h, mlo, mhi, xp, xacc,
                  wh1, bh1, wh2, bh2, g, b,
                  hn_o, xp_o):
    f32 = jnp.float32
    hv = h[...]
    cat = jnp.concatenate([hv, mlo[...], mhi[...]], axis=1)
    pre = jnp.dot(cat, wh1[...], preferred_element_type=f32) + bh1[...]
    u = jnp.dot(_silu(pre), wh2[...], preferred_element_type=f32) + bh2[...]
    hn = hv + u
    mu = jnp.mean(hn, axis=1, keepdims=True)
    var = jnp.mean((hn - mu) * (hn - mu), axis=1, keepdims=True)
    hn_o[...] = (hn - mu) * lax.rsqrt(var + 1e-5) * g[...] + b[...]
    xp_o[...] = xp[...] + xacc[...]


def _node_call(h, mlo, mhi, xp, xacc, wh1, bh1, wh2, bh2, g, b):
    grid = (_N // _BN,)
    row = lambda i: (i, 0)
    full = lambda i: (0, 0)
    return pl.pallas_call(
        _tc_node_body,
        grid=grid,
        in_specs=[
            pl.BlockSpec((_BN, _ND), row),
            pl.BlockSpec((_BN, _ND), row),
            pl.BlockSpec((_BN, _ND), row),
            pl.BlockSpec((_BN, _XP), row),
            pl.BlockSpec((_BN, _XP), row),
            pl.BlockSpec((_ND + _HD, _HD), full),
            pl.BlockSpec((1, _HD), full),
            pl.BlockSpec((_HD, _ND), full),
            pl.BlockSpec((1, _ND), full),
            pl.BlockSpec((1, _ND), full),
            pl.BlockSpec((1, _ND), full),
        ],
        out_specs=[
            pl.BlockSpec((_BN, _ND), row),
            pl.BlockSpec((_BN, _XP), row),
        ],
        out_shape=[
            jax.ShapeDtypeStruct((_N, _ND), jnp.float32),
            jax.ShapeDtypeStruct((_N, _XP), jnp.float32),
        ],
        compiler_params=pltpu.CompilerParams(
            dimension_semantics=("arbitrary",),
        ),
    )(h, mlo, mhi, xp, xacc, wh1, bh1, wh2, bh2, g, b)


def kernel(h, x, edge_index, edge_attr, params):
    ei = edge_index.astype(jnp.int32)
    ii = ei[1]
    jj = ei[0]
    xp = jnp.pad(x.astype(jnp.float32), ((0, 0), (0, _XP - 3)))
    z = jnp.zeros((_NPS, _ND), jnp.float32)
    zx = jnp.zeros((_NPS, _XP), jnp.float32)
    for p in params:
        hi, hj, xi, xj = _gather_call(h, xp, ii, jj)
        mlo, mhi, wd = _edge_call(
            hi, hj, xi, xj, edge_attr,
            p["e1"]["w"], p["e1"]["b"][None],
            p["e2"]["w"], p["e2"]["b"][None],
            p["x1"]["w"], p["x1"]["b"][None], p["x2"]["w"],
        )
        silo, sihi, xacc = _scatter_call(mlo, mhi, wd, ii, z, zx)
        h, xp = _node_call(
            h, silo, sihi, xp, xacc,
            p["h1"]["w"], p["h1"]["b"][None],
            p["h2"]["w"], p["h2"]["b"][None], p["ln_g"][None], p["ln_b"][None],
        )
    return (h, xp[:, :3])
